# baseline (speedup 1.0000x reference)
import functools

import jax
import jax.numpy as jnp
from jax import lax
from jax.experimental import pallas as pl
from jax.experimental.pallas import tpu as pltpu
from jax.experimental.pallas import tpu_sc as plsc

N = 160000
D = 256
DH = 128
DOUT = 512
S = 10000
NTILES = 16
CH = 128
ROWS_PER_TILE = N // NTILES
NFULL = ROWS_PER_TILE // CH
TAIL = ROWS_PER_TILE - NFULL * CH
HIST_SPLIT = NFULL // 2
SEG_PER_TILE = 624
SEG_REM = S - NTILES * SEG_PER_TILE

_mesh = plsc.VectorSubcoreMesh(core_axis_name="c", subcore_axis_name="s")


@functools.partial(
    pl.kernel,
    out_type=(
        jax.ShapeDtypeStruct((S, D), jnp.float32),
        jax.ShapeDtypeStruct((32, S), jnp.float32),
    ),
    mesh=_mesh,
    compiler_params=pltpu.CompilerParams(needs_layout_passes=False),
    scratch_types=[
        pltpu.VMEM((2, CH, DH), jnp.float32),
        pltpu.VMEM((2, CH), jnp.int32),
        pltpu.VMEM((16,), jnp.int32),
        pltpu.VMEM((S,), jnp.float32),
        pltpu.VMEM_SHARED((S, DH), jnp.float32),
        pltpu.SemaphoreType.DMA,
        pltpu.SemaphoreType.DMA,
        pltpu.SemaphoreType.DMA,
        pltpu.SemaphoreType.DMA,
        pltpu.SemaphoreType.DMA,
        pltpu.SemaphoreType.DMA,
    ],
)
def _segsum_sc(x_hbm, ids_hbm, seg_hbm, cnt_hbm, rowbuf, idbuf, idtail, hist,
               acc, sx0, sx1, si0, si1, ss0, ss1):
    c = lax.axis_index("c")
    t = lax.axis_index("s")
    col0 = c * DH
    base = t * ROWS_PER_TILE
    seg0 = t * SEG_PER_TILE
    zero16 = jnp.zeros((16,), jnp.float32)
    one16 = jnp.ones((16,), jnp.float32)

    sx = (sx0, sx1)
    si = (si0, si1)

    def _fill0(r, carry):
        for j in range(DH // 16):
            rowbuf[0, r, pl.ds(j * 16, 16)] = zero16
        return carry

    lax.fori_loop(0, CH, _fill0, 0)

    def _zhist(r, carry):
        hist[pl.ds(r * 16, 16)] = zero16
        return carry

    lax.fori_loop(0, S // 16, _zhist, 0)

    for k in range(4):
        pltpu.sync_copy(rowbuf.at[0], acc.at[pl.ds(seg0 + k * CH, CH)])
    pltpu.sync_copy(rowbuf.at[0, pl.ds(0, SEG_PER_TILE - 4 * CH)],
                    acc.at[pl.ds(seg0 + 4 * CH, SEG_PER_TILE - 4 * CH)])

    @pl.when(t == NTILES - 1)
    def _():
        pltpu.sync_copy(rowbuf.at[0, pl.ds(0, SEG_REM)],
                        acc.at[pl.ds(S - SEG_REM, SEG_REM)])

    def _start_loads(j, b):
        r0 = base + j * CH
        pltpu.async_copy(ids_hbm.at[pl.ds(r0, CH)], idbuf.at[b], si[b])
        pltpu.async_copy(x_hbm.at[pl.ds(r0, CH), pl.ds(col0, DH)],
                         rowbuf.at[b], sx[b])

    def _wait_loads(b):
        r0 = base
        pltpu.make_async_copy(ids_hbm.at[pl.ds(r0, CH)], idbuf.at[b],
                              si[b]).wait()
        pltpu.make_async_copy(x_hbm.at[pl.ds(r0, CH), pl.ds(col0, DH)],
                              rowbuf.at[b], sx[b]).wait()

    _start_loads(0, 0)
    _start_loads(1, 1)
    plsc.subcore_barrier()

    ss = (ss0, ss1)

    def _loop(g, carry):
        descs = []
        for b in range(2):
            _wait_loads(b)
            descs.append(pltpu.async_copy(rowbuf.at[b], acc.at[idbuf.at[b]],
                                          ss[b], add=True))
        for b in range(2):
            j = 2 * g + b

            @pl.when((j < HIST_SPLIT) == (c == 0))
            def _():
                for k in range(CH // 16):
                    iv = idbuf[b, pl.ds(k * 16, 16)]
                    plsc.addupdate_scatter(hist, [iv], one16)

        for b in range(2):
            j = 2 * g + b
            descs[b].wait()
            _start_loads(jnp.minimum(j + 2, NFULL - 1), b)
        return carry

    lax.fori_loop(0, NFULL // 2, _loop, 0)

    _wait_loads(0)
    _wait_loads(1)

    r0 = base + NFULL * CH
    pltpu.sync_copy(ids_hbm.at[pl.ds(r0, TAIL)], idtail)
    pltpu.sync_copy(x_hbm.at[pl.ds(r0, TAIL), pl.ds(col0, DH)],
                    rowbuf.at[0, pl.ds(0, TAIL)])
    pltpu.sync_copy(rowbuf.at[0, pl.ds(0, TAIL)], acc.at[idtail], add=True)

    @pl.when(c == 1)
    def _():
        iv = idtail[...]
        plsc.addupdate_scatter(hist, [iv], one16)

    plsc.subcore_barrier()

    pltpu.sync_copy(acc.at[pl.ds(seg0, SEG_PER_TILE)],
                    seg_hbm.at[pl.ds(seg0, SEG_PER_TILE), pl.ds(col0, DH)])
    pltpu.sync_copy(hist, cnt_hbm.at[c * NTILES + t])

    @pl.when(t == NTILES - 1)
    def _():
        pltpu.sync_copy(acc.at[pl.ds(S - SEG_REM, SEG_REM)],
                        seg_hbm.at[pl.ds(S - SEG_REM, SEG_REM), pl.ds(col0, DH)])


BM = 2000


def _project_body(s_ref, w_ref, cnt_ref, b_ref, o_ref):
    acc = lax.dot_general(
        s_ref[...], w_ref[...], (((1,), (1,)), ((), ())),
        preferred_element_type=jnp.float32,
        precision=lax.Precision.HIGHEST,
    )
    bias = lax.dot_general(
        cnt_ref[...], b_ref[...], (((1,), (0,)), ((), ())),
        preferred_element_type=jnp.float32,
        precision=lax.Precision.HIGHEST,
    )
    o_ref[...] = acc + bias


_project = pl.pallas_call(
    _project_body,
    grid=(S // BM,),
    in_specs=[
        pl.BlockSpec((BM, D), lambda i: (i, 0)),
        pl.BlockSpec((DOUT, D), lambda i: (0, 0)),
        pl.BlockSpec((BM, 32), lambda i: (i, 0)),
        pl.BlockSpec((32, DOUT), lambda i: (0, 0)),
    ],
    out_specs=pl.BlockSpec((BM, DOUT), lambda i: (i, 0)),
    out_shape=jax.ShapeDtypeStruct((S, DOUT), jnp.float32),
)


def kernel(x, signal_belongings, W, b):
    ids = signal_belongings.astype(jnp.int32)
    seg, cnt32 = _segsum_sc(x, ids)
    b_rep = jnp.broadcast_to(b.reshape(1, DOUT), (32, DOUT))
    return _project(seg, W, cnt32.T, b_rep)


# speedup vs baseline: 1.3081x; 1.3081x over previous
import functools

import jax
import jax.numpy as jnp
from jax import lax
from jax.experimental import pallas as pl
from jax.experimental.pallas import tpu as pltpu
from jax.experimental.pallas import tpu_sc as plsc

N = 160000
D = 256
DH = 128
DOUT = 512
S = 10000
NTILES = 16
CH = 128
ROWS_PER_TILE = N // NTILES
NFULL = ROWS_PER_TILE // CH
TAIL = ROWS_PER_TILE - NFULL * CH
HIST_SPLIT = NFULL // 2
SEG_PER_TILE = 624
SEG_REM = S - NTILES * SEG_PER_TILE

_mesh = plsc.VectorSubcoreMesh(core_axis_name="c", subcore_axis_name="s")


@functools.partial(
    pl.kernel,
    out_type=(
        jax.ShapeDtypeStruct((S, D), jnp.float32),
        jax.ShapeDtypeStruct((32, S), jnp.float32),
    ),
    mesh=_mesh,
    compiler_params=pltpu.CompilerParams(needs_layout_passes=False),
    scratch_types=[
        pltpu.VMEM((2, CH, DH), jnp.float32),
        pltpu.VMEM((2, CH), jnp.int32),
        pltpu.VMEM((16,), jnp.int32),
        pltpu.VMEM((S,), jnp.float32),
        pltpu.VMEM_SHARED((S, DH), jnp.float32),
        pltpu.SemaphoreType.DMA,
        pltpu.SemaphoreType.DMA,
        pltpu.SemaphoreType.DMA,
        pltpu.SemaphoreType.DMA,
        pltpu.SemaphoreType.DMA,
        pltpu.SemaphoreType.DMA,
    ],
)
def _segsum_sc(x_hbm, ids_hbm, seg_hbm, cnt_hbm, rowbuf, idbuf, idtail, hist,
               acc, sx0, sx1, si0, si1, ss0, ss1):
    c = lax.axis_index("c")
    t = lax.axis_index("s")
    col0 = c * DH
    base = t * ROWS_PER_TILE
    seg0 = t * SEG_PER_TILE
    zero16 = jnp.zeros((16,), jnp.float32)
    one16 = jnp.ones((16,), jnp.float32)

    sx = (sx0, sx1)
    si = (si0, si1)

    def _fill0(r, carry):
        for j in range(DH // 16):
            rowbuf[0, r, pl.ds(j * 16, 16)] = zero16
        return carry

    lax.fori_loop(0, CH, _fill0, 0)

    def _zhist(r, carry):
        hist[pl.ds(r * 16, 16)] = zero16
        return carry

    lax.fori_loop(0, S // 16, _zhist, 0)

    for k in range(4):
        pltpu.sync_copy(rowbuf.at[0], acc.at[pl.ds(seg0 + k * CH, CH)])
    pltpu.sync_copy(rowbuf.at[0, pl.ds(0, SEG_PER_TILE - 4 * CH)],
                    acc.at[pl.ds(seg0 + 4 * CH, SEG_PER_TILE - 4 * CH)])

    @pl.when(t == NTILES - 1)
    def _():
        pltpu.sync_copy(rowbuf.at[0, pl.ds(0, SEG_REM)],
                        acc.at[pl.ds(S - SEG_REM, SEG_REM)])

    def _start_loads(j, b):
        r0 = base + j * CH
        pltpu.async_copy(ids_hbm.at[pl.ds(r0, CH)], idbuf.at[b], si[b])
        pltpu.async_copy(x_hbm.at[pl.ds(r0, CH), pl.ds(col0, DH)],
                         rowbuf.at[b], sx[b])

    def _wait_loads(b):
        r0 = base
        pltpu.make_async_copy(ids_hbm.at[pl.ds(r0, CH)], idbuf.at[b],
                              si[b]).wait()
        pltpu.make_async_copy(x_hbm.at[pl.ds(r0, CH), pl.ds(col0, DH)],
                              rowbuf.at[b], sx[b]).wait()

    _start_loads(0, 0)
    _start_loads(1, 1)
    plsc.subcore_barrier()

    def _loop(g, carry):
        for b in range(2):
            j = 2 * g + b
            _wait_loads(b)
            pltpu.sync_copy(rowbuf.at[b], acc.at[idbuf.at[b]], add=True)

            @pl.when((j < HIST_SPLIT) == (c == 0))
            def _():
                for k in range(CH // 16):
                    iv = idbuf[b, pl.ds(k * 16, 16)]
                    plsc.addupdate_scatter(hist, [iv], one16)

            _start_loads(jnp.minimum(j + 2, NFULL - 1), b)
        return carry

    lax.fori_loop(0, NFULL // 2, _loop, 0)

    _wait_loads(0)
    _wait_loads(1)

    r0 = base + NFULL * CH
    pltpu.sync_copy(ids_hbm.at[pl.ds(r0, TAIL)], idtail)
    pltpu.sync_copy(x_hbm.at[pl.ds(r0, TAIL), pl.ds(col0, DH)],
                    rowbuf.at[0, pl.ds(0, TAIL)])
    pltpu.sync_copy(rowbuf.at[0, pl.ds(0, TAIL)], acc.at[idtail], add=True)

    @pl.when(c == 1)
    def _():
        iv = idtail[...]
        plsc.addupdate_scatter(hist, [iv], one16)

    plsc.subcore_barrier()

    pltpu.sync_copy(acc.at[pl.ds(seg0, SEG_PER_TILE)],
                    seg_hbm.at[pl.ds(seg0, SEG_PER_TILE), pl.ds(col0, DH)])
    pltpu.sync_copy(hist, cnt_hbm.at[c * NTILES + t])

    @pl.when(t == NTILES - 1)
    def _():
        pltpu.sync_copy(acc.at[pl.ds(S - SEG_REM, SEG_REM)],
                        seg_hbm.at[pl.ds(S - SEG_REM, SEG_REM), pl.ds(col0, DH)])


BM = 2000


def _project_body(s_ref, w_ref, cnt_ref, b_ref, o_ref):
    acc = lax.dot_general(
        s_ref[...], w_ref[...], (((1,), (1,)), ((), ())),
        preferred_element_type=jnp.float32,
    )
    b_rep = jnp.broadcast_to(b_ref[...], (32, DOUT))
    bias = lax.dot_general(
        cnt_ref[...], b_rep, (((1,), (0,)), ((), ())),
        preferred_element_type=jnp.float32,
        precision=lax.Precision.HIGHEST,
    )
    o_ref[...] = acc + bias


_project = pl.pallas_call(
    _project_body,
    grid=(S // BM,),
    in_specs=[
        pl.BlockSpec((BM, D), lambda i: (i, 0)),
        pl.BlockSpec((DOUT, D), lambda i: (0, 0)),
        pl.BlockSpec((BM, 32), lambda i: (i, 0)),
        pl.BlockSpec((1, DOUT), lambda i: (0, 0)),
    ],
    out_specs=pl.BlockSpec((BM, DOUT), lambda i: (i, 0)),
    out_shape=jax.ShapeDtypeStruct((S, DOUT), jnp.float32),
)


def kernel(x, signal_belongings, W, b):
    ids = signal_belongings.astype(jnp.int32)
    seg, cnt32 = _segsum_sc(x, ids)
    return _project(seg, W, cnt32.T, b.reshape(1, DOUT))
